# baseline (speedup 1.0000x reference)
import jax
import jax.numpy as jnp
from jax import lax
from jax.experimental import pallas as pl
from jax.experimental.pallas import tpu as pltpu
from jax.experimental.pallas import tpu_sc as plsc

N_NODES = 10000
N_EDGES = 320000
D = 128
DH = D // 2
OUT = 40

NC, NS = 2, 16
E_TILE = N_EDGES // NS
CH = 80
NCH = E_TILE // CH
NBUF = 5
PF = NBUF - 2
DEGW = 16

_mesh = plsc.VectorSubcoreMesh(core_axis_name="c", subcore_axis_name="s")


def _sc_agg_body(h_hbm, src_hbm, dst_hbm, zagg_hbm, agg_out,
                 src_v, dst_v, rows_v, agg_sh, *sems):
    c = lax.axis_index("c")
    s = lax.axis_index("s")

    @pl.when(s == 0)
    def _():
        pltpu.sync_copy(zagg_hbm, agg_sh)

    pltpu.sync_copy(src_hbm.at[s], src_v)
    pltpu.sync_copy(dst_hbm.at[s], dst_v)
    plsc.subcore_barrier()

    h_c = h_hbm.at[c]
    gsem = sems[:NBUF]
    ssem = sems[NBUF:]

    for b in range(PF):
        pltpu.async_copy(h_c.at[src_v.at[b]], rows_v.at[b], gsem[b])

    def group(i, carry):
        for k in range(NBUF):
            j = NBUF * i + k
            kp = (k + PF) % NBUF
            pltpu.make_async_copy(h_c.at[src_v.at[j]], rows_v.at[k],
                                  gsem[k]).wait()
            pltpu.async_copy(rows_v.at[k], agg_sh.at[dst_v.at[j]],
                             ssem[k], add=True)

            @pl.when(j + PF < NCH)
            def _():
                @pl.when(j >= 2)
                def _():
                    pltpu.make_async_copy(
                        rows_v.at[kp], agg_sh.at[dst_v.at[j - 2]],
                        ssem[kp]).wait()

                pltpu.async_copy(h_c.at[src_v.at[j + PF]],
                                 rows_v.at[kp], gsem[kp])
        return carry

    lax.fori_loop(0, NCH // NBUF, group, None)

    for j in range(NCH - NBUF, NCH):
        b = j % NBUF
        pltpu.make_async_copy(rows_v.at[b], agg_sh.at[dst_v.at[j]],
                              ssem[b]).wait()

    plsc.subcore_barrier()

    @pl.when(s == 0)
    def _():
        pltpu.sync_copy(agg_sh, agg_out.at[c])


_sc_agg = pl.kernel(
    _sc_agg_body,
    out_type=(jax.ShapeDtypeStruct((NC, N_NODES, DH), jnp.float32),),
    mesh=_mesh,
    compiler_params=pltpu.CompilerParams(use_tc_tiling_on_sc=False),
    scratch_types=[
        pltpu.VMEM((NCH, CH), jnp.int32),
        pltpu.VMEM((NCH, CH), jnp.int32),
        pltpu.VMEM((NBUF, CH, DH), jnp.float32),
        pltpu.VMEM_SHARED((N_NODES, DH), jnp.float32),
    ] + [pltpu.SemaphoreType.DMA] * (2 * NBUF),
)


def _sc_deg_body(dst_hbm, zdeg_hbm, ones_hbm, deg_out,
                 dst_v, ones_v, deg_sh):
    c = lax.axis_index("c")
    s = lax.axis_index("s")

    @pl.when(s == 0)
    def _():
        pltpu.sync_copy(zdeg_hbm, deg_sh)

    pltpu.sync_copy(ones_hbm, ones_v)
    pltpu.sync_copy(dst_hbm.at[s], dst_v)
    plsc.subcore_barrier()

    half = NCH // 2

    def step(j, carry):
        pltpu.sync_copy(ones_v, deg_sh.at[dst_v.at[c * half + j]], add=True)
        return carry

    lax.fori_loop(0, half, step, None)
    plsc.subcore_barrier()

    @pl.when(s == 0)
    def _():
        pltpu.sync_copy(deg_sh, deg_out.at[c])


_sc_deg = pl.kernel(
    _sc_deg_body,
    out_type=(jax.ShapeDtypeStruct((NC, N_NODES, DEGW), jnp.float32),),
    mesh=_mesh,
    compiler_params=pltpu.CompilerParams(use_tc_tiling_on_sc=False),
    scratch_types=[
        pltpu.VMEM((NCH, CH), jnp.int32),
        pltpu.VMEM((CH, DEGW), jnp.float32),
        pltpu.VMEM_SHARED((N_NODES, DEGW), jnp.float32),
    ],
)


RB = 2000


def _tc_layer_body(aggp_ref, degp_ref, w_ref, b_ref, out_ref):
    agg = jnp.concatenate([aggp_ref[0], aggp_ref[1]], axis=1)
    deg = degp_ref[0] + degp_ref[1]
    deg0 = jnp.maximum(deg[:, 0:1], 1.0)
    h = agg / deg0
    acc = jnp.dot(h, w_ref[...], preferred_element_type=jnp.float32)
    h = jnp.maximum(acc + b_ref[...], 0.0)
    out_ref[0] = h[:, :DH]
    out_ref[1] = h[:, DH:]


def _tc_layer(aggp, degp, w, b):
    grid = N_NODES // RB
    return pl.pallas_call(
        _tc_layer_body,
        grid=(grid,),
        in_specs=[
            pl.BlockSpec((NC, RB, DH), lambda i: (0, i, 0)),
            pl.BlockSpec((NC, RB, DEGW), lambda i: (0, i, 0)),
            pl.BlockSpec((D, D), lambda i: (0, 0)),
            pl.BlockSpec((1, D), lambda i: (0, 0)),
        ],
        out_specs=pl.BlockSpec((NC, RB, DH), lambda i: (0, i, 0)),
        out_shape=jax.ShapeDtypeStruct((NC, N_NODES, DH), jnp.float32),
    )(aggp, degp, w, b)


def _tc_final_body(aggp_ref, degp_ref, w3_ref, b3_ref, wf1_ref, bf1_ref,
                   wf2_ref, bf2_ref, out_ref):
    agg = jnp.concatenate([aggp_ref[0], aggp_ref[1]], axis=1)
    deg = degp_ref[0] + degp_ref[1]
    deg0 = jnp.maximum(deg[:, 0:1], 1.0)
    h = agg / deg0
    h = jnp.maximum(
        jnp.dot(h, w3_ref[...], preferred_element_type=jnp.float32)
        + b3_ref[...], 0.0)
    h = jnp.maximum(
        jnp.dot(h, wf1_ref[...], preferred_element_type=jnp.float32)
        + bf1_ref[...], 0.0)
    logits = (jnp.dot(h, wf2_ref[...], preferred_element_type=jnp.float32)
              + bf2_ref[...])
    col = lax.broadcasted_iota(jnp.int32, logits.shape, 1)
    logits = jnp.where(col < OUT, logits, -1e30)
    m = jnp.max(logits, axis=1, keepdims=True)
    e = jnp.exp(logits - m)
    lse = jnp.log(jnp.sum(e, axis=1, keepdims=True))
    out_ref[...] = logits - m - lse


def _tc_final(aggp, degp, w3, b3, wf1, bf1, wf2p, bf2p):
    grid = N_NODES // RB
    full = lambda r, c_: pl.BlockSpec((r, c_), lambda i: (0, 0))
    return pl.pallas_call(
        _tc_final_body,
        grid=(grid,),
        in_specs=[
            pl.BlockSpec((NC, RB, DH), lambda i: (0, i, 0)),
            pl.BlockSpec((NC, RB, DEGW), lambda i: (0, i, 0)),
            full(D, D), full(1, D),
            full(D, D), full(1, D),
            full(D, D), full(1, D),
        ],
        out_specs=pl.BlockSpec((RB, D), lambda i: (i, 0)),
        out_shape=jax.ShapeDtypeStruct((N_NODES, D), jnp.float32),
    )(aggp, degp, w3, b3, wf1, bf1, wf2p, bf2p)


def kernel(x, edge_index, W1, b1, W2, b2, W3, b3, Wf1, bf1, Wf2, bf2):
    ei = edge_index.astype(jnp.int32)
    src = ei[0].reshape(NS, NCH, CH)
    dst = ei[1].reshape(NS, NCH, CH)
    zagg = jnp.zeros((N_NODES, DH), jnp.float32)
    zdeg = jnp.zeros((N_NODES, DEGW), jnp.float32)
    ones = jnp.ones((CH, DEGW), jnp.float32)
    x2 = jnp.stack([x[:, :DH], x[:, DH:]])

    (degp,) = _sc_deg(dst, zdeg, ones)
    (aggp,) = _sc_agg(x2, src, dst, zagg)
    h2 = _tc_layer(aggp, degp, W1, b1.reshape(1, D))
    (aggp,) = _sc_agg(h2, src, dst, zagg)
    h2 = _tc_layer(aggp, degp, W2, b2.reshape(1, D))
    (aggp,) = _sc_agg(h2, src, dst, zagg)

    wf2p = jnp.zeros((D, D), jnp.float32).at[:, :OUT].set(Wf2)
    bf2p = jnp.zeros((1, D), jnp.float32).at[0, :OUT].set(bf2)
    out = _tc_final(aggp, degp, W3, b3.reshape(1, D),
                    Wf1, bf1.reshape(1, D), wf2p, bf2p)
    return out[:, :OUT]


# speedup vs baseline: 1.0793x; 1.0793x over previous
import jax
import jax.numpy as jnp
from jax import lax
from jax.experimental import pallas as pl
from jax.experimental.pallas import tpu as pltpu
from jax.experimental.pallas import tpu_sc as plsc

N_NODES = 10000
N_EDGES = 320000
D = 128
DH = D // 2
OUT = 40

NC, NS = 2, 16
E_TILE = N_EDGES // NS
CH = 80
NCH = E_TILE // CH
NBUF = 5
DEGW = 16

_mesh = plsc.VectorSubcoreMesh(core_axis_name="c", subcore_axis_name="s")


def _sc_agg_body(h_hbm, src_hbm, dst_hbm, zagg_hbm, agg_out,
                 src_v, dst_v, rows_v, agg_sh, *sems):
    c = lax.axis_index("c")
    s = lax.axis_index("s")

    @pl.when(s == 0)
    def _():
        pltpu.sync_copy(zagg_hbm, agg_sh)

    pltpu.sync_copy(src_hbm.at[s], src_v)
    pltpu.sync_copy(dst_hbm.at[s], dst_v)
    plsc.subcore_barrier()

    h_c = h_hbm.at[c]
    gsem = sems

    for b in range(NBUF - 1):
        pltpu.async_copy(h_c.at[src_v.at[b]], rows_v.at[b], gsem[b])

    def group(i, carry):
        for k in range(NBUF):
            j = NBUF * i + k
            pltpu.make_async_copy(h_c.at[src_v.at[j]], rows_v.at[k],
                                  gsem[k]).wait()
            kn = (k + NBUF - 1) % NBUF

            @pl.when(j + NBUF - 1 < NCH)
            def _():
                pltpu.async_copy(h_c.at[src_v.at[j + NBUF - 1]],
                                 rows_v.at[kn], gsem[kn])

            pltpu.sync_copy(rows_v.at[k], agg_sh.at[dst_v.at[j]], add=True)
        return carry

    lax.fori_loop(0, NCH // NBUF, group, None)

    plsc.subcore_barrier()

    @pl.when(s == 0)
    def _():
        pltpu.sync_copy(agg_sh, agg_out.at[c])


_sc_agg = pl.kernel(
    _sc_agg_body,
    out_type=(jax.ShapeDtypeStruct((NC, N_NODES, DH), jnp.float32),),
    mesh=_mesh,
    compiler_params=pltpu.CompilerParams(use_tc_tiling_on_sc=False),
    scratch_types=[
        pltpu.VMEM((NCH, CH), jnp.int32),
        pltpu.VMEM((NCH, CH), jnp.int32),
        pltpu.VMEM((NBUF, CH, DH), jnp.float32),
        pltpu.VMEM_SHARED((N_NODES, DH), jnp.float32),
    ] + [pltpu.SemaphoreType.DMA] * NBUF,
)


def _sc_deg_body(dst_hbm, zdeg_hbm, ones_hbm, deg_out,
                 dst_v, ones_v, deg_sh):
    c = lax.axis_index("c")
    s = lax.axis_index("s")

    @pl.when(s == 0)
    def _():
        pltpu.sync_copy(zdeg_hbm, deg_sh)

    pltpu.sync_copy(ones_hbm, ones_v)
    pltpu.sync_copy(dst_hbm.at[s], dst_v)
    plsc.subcore_barrier()

    half = NCH // 2

    def step(j, carry):
        pltpu.sync_copy(ones_v, deg_sh.at[dst_v.at[c * half + j]], add=True)
        return carry

    lax.fori_loop(0, half, step, None)
    plsc.subcore_barrier()

    @pl.when(s == 0)
    def _():
        pltpu.sync_copy(deg_sh, deg_out.at[c])


_sc_deg = pl.kernel(
    _sc_deg_body,
    out_type=(jax.ShapeDtypeStruct((NC, N_NODES, DEGW), jnp.float32),),
    mesh=_mesh,
    compiler_params=pltpu.CompilerParams(use_tc_tiling_on_sc=False),
    scratch_types=[
        pltpu.VMEM((NCH, CH), jnp.int32),
        pltpu.VMEM((CH, DEGW), jnp.float32),
        pltpu.VMEM_SHARED((N_NODES, DEGW), jnp.float32),
    ],
)


RB = 2000


def _tc_layer_body(aggp_ref, degp_ref, w_ref, b_ref, out_ref):
    agg = jnp.concatenate([aggp_ref[0], aggp_ref[1]], axis=1)
    deg = degp_ref[0] + degp_ref[1]
    deg0 = jnp.maximum(deg[:, 0:1], 1.0)
    h = agg / deg0
    acc = jnp.dot(h, w_ref[...], preferred_element_type=jnp.float32)
    h = jnp.maximum(acc + b_ref[...], 0.0)
    out_ref[0] = h[:, :DH]
    out_ref[1] = h[:, DH:]


def _tc_layer(aggp, degp, w, b):
    grid = N_NODES // RB
    return pl.pallas_call(
        _tc_layer_body,
        grid=(grid,),
        in_specs=[
            pl.BlockSpec((NC, RB, DH), lambda i: (0, i, 0)),
            pl.BlockSpec((NC, RB, DEGW), lambda i: (0, i, 0)),
            pl.BlockSpec((D, D), lambda i: (0, 0)),
            pl.BlockSpec((1, D), lambda i: (0, 0)),
        ],
        out_specs=pl.BlockSpec((NC, RB, DH), lambda i: (0, i, 0)),
        out_shape=jax.ShapeDtypeStruct((NC, N_NODES, DH), jnp.float32),
    )(aggp, degp, w, b)


def _tc_final_body(aggp_ref, degp_ref, w3_ref, b3_ref, wf1_ref, bf1_ref,
                   wf2_ref, bf2_ref, out_ref):
    agg = jnp.concatenate([aggp_ref[0], aggp_ref[1]], axis=1)
    deg = degp_ref[0] + degp_ref[1]
    deg0 = jnp.maximum(deg[:, 0:1], 1.0)
    h = agg / deg0
    h = jnp.maximum(
        jnp.dot(h, w3_ref[...], preferred_element_type=jnp.float32)
        + b3_ref[...], 0.0)
    h = jnp.maximum(
        jnp.dot(h, wf1_ref[...], preferred_element_type=jnp.float32)
        + bf1_ref[...], 0.0)
    logits = (jnp.dot(h, wf2_ref[...], preferred_element_type=jnp.float32)
              + bf2_ref[...])
    col = lax.broadcasted_iota(jnp.int32, logits.shape, 1)
    logits = jnp.where(col < OUT, logits, -1e30)
    m = jnp.max(logits, axis=1, keepdims=True)
    e = jnp.exp(logits - m)
    lse = jnp.log(jnp.sum(e, axis=1, keepdims=True))
    res = logits - m - lse
    out_ref[...] = res[:, :OUT]


def _tc_final(aggp, degp, w3, b3, wf1, bf1, wf2p, bf2p):
    grid = N_NODES // RB
    full = lambda r, c_: pl.BlockSpec((r, c_), lambda i: (0, 0))
    return pl.pallas_call(
        _tc_final_body,
        grid=(grid,),
        in_specs=[
            pl.BlockSpec((NC, RB, DH), lambda i: (0, i, 0)),
            pl.BlockSpec((NC, RB, DEGW), lambda i: (0, i, 0)),
            full(D, D), full(1, D),
            full(D, D), full(1, D),
            full(D, D), full(1, D),
        ],
        out_specs=pl.BlockSpec((RB, OUT), lambda i: (i, 0)),
        out_shape=jax.ShapeDtypeStruct((N_NODES, OUT), jnp.float32),
    )(aggp, degp, w3, b3, wf1, bf1, wf2p, bf2p)


def kernel(x, edge_index, W1, b1, W2, b2, W3, b3, Wf1, bf1, Wf2, bf2):
    ei = edge_index.astype(jnp.int32)
    src = ei[0].reshape(NS, NCH, CH)
    dst = ei[1].reshape(NS, NCH, CH)
    zagg = jnp.zeros((N_NODES, DH), jnp.float32)
    zdeg = jnp.zeros((N_NODES, DEGW), jnp.float32)
    ones = jnp.ones((CH, DEGW), jnp.float32)
    x2 = jnp.stack([x[:, :DH], x[:, DH:]])

    (degp,) = _sc_deg(dst, zdeg, ones)
    x2, degp = jax.lax.optimization_barrier((x2, degp))
    (aggp,) = _sc_agg(x2, src, dst, zagg)
    h2 = _tc_layer(aggp, degp, W1, b1.reshape(1, D))
    (aggp,) = _sc_agg(h2, src, dst, zagg)
    h2 = _tc_layer(aggp, degp, W2, b2.reshape(1, D))
    (aggp,) = _sc_agg(h2, src, dst, zagg)

    wf2p = jnp.zeros((D, D), jnp.float32).at[:, :OUT].set(Wf2)
    bf2p = jnp.zeros((1, D), jnp.float32).at[0, :OUT].set(bf2)
    return _tc_final(aggp, degp, W3, b3.reshape(1, D),
                     Wf1, bf1.reshape(1, D), wf2p, bf2p)


# speedup vs baseline: 1.1057x; 1.0245x over previous
import jax
import jax.numpy as jnp
from jax import lax
from jax.experimental import pallas as pl
from jax.experimental.pallas import tpu as pltpu
from jax.experimental.pallas import tpu_sc as plsc

N_NODES = 10000
N_EDGES = 320000
D = 128
DH = D // 2
OUT = 40

NC, NS = 2, 16
E_TILE = N_EDGES // NS
CH = 80
NCH = E_TILE // CH
NBUF = 5
DEGW = 16

_mesh = plsc.VectorSubcoreMesh(core_axis_name="c", subcore_axis_name="s")


def _sc_agg_body(h_hbm, src_hbm, dst_hbm, zagg_hbm, agg_out,
                 src_v, dst_v, rows_v, agg_sh, *sems):
    c = lax.axis_index("c")
    s = lax.axis_index("s")

    @pl.when(s == 0)
    def _():
        pltpu.sync_copy(zagg_hbm, agg_sh)

    pltpu.sync_copy(src_hbm.at[s], src_v)
    pltpu.sync_copy(dst_hbm.at[s], dst_v)
    plsc.subcore_barrier()

    h_c = h_hbm.at[c]
    gsem = sems

    for b in range(NBUF - 1):
        pltpu.async_copy(h_c.at[src_v.at[b]], rows_v.at[b], gsem[b])

    def group(i, carry):
        for k in range(NBUF):
            j = NBUF * i + k
            pltpu.make_async_copy(h_c.at[src_v.at[j]], rows_v.at[k],
                                  gsem[k]).wait()
            kn = (k + NBUF - 1) % NBUF

            @pl.when(j + NBUF - 1 < NCH)
            def _():
                pltpu.async_copy(h_c.at[src_v.at[j + NBUF - 1]],
                                 rows_v.at[kn], gsem[kn])

            pltpu.sync_copy(rows_v.at[k], agg_sh.at[dst_v.at[j]], add=True)
        return carry

    lax.fori_loop(0, NCH // NBUF, group, None)

    plsc.subcore_barrier()

    @pl.when(s == 0)
    def _():
        pltpu.sync_copy(agg_sh, agg_out.at[c])


_sc_agg = pl.kernel(
    _sc_agg_body,
    out_type=(jax.ShapeDtypeStruct((NC, N_NODES, DH), jnp.float32),),
    mesh=_mesh,
    compiler_params=pltpu.CompilerParams(use_tc_tiling_on_sc=False),
    scratch_types=[
        pltpu.VMEM((NCH, CH), jnp.int32),
        pltpu.VMEM((NCH, CH), jnp.int32),
        pltpu.VMEM((NBUF, CH, DH), jnp.float32),
        pltpu.VMEM_SHARED((N_NODES, DH), jnp.float32),
    ] + [pltpu.SemaphoreType.DMA] * NBUF,
)


def _sc_deg_body(dst_hbm, zdeg_hbm, ones_hbm, deg_out,
                 dst_v, ones_v, deg_sh):
    c = lax.axis_index("c")
    s = lax.axis_index("s")

    @pl.when(s == 0)
    def _():
        pltpu.sync_copy(zdeg_hbm, deg_sh)

    pltpu.sync_copy(ones_hbm, ones_v)
    pltpu.sync_copy(dst_hbm.at[s], dst_v)
    plsc.subcore_barrier()

    half = NCH // 2

    def step(j, carry):
        pltpu.sync_copy(ones_v, deg_sh.at[dst_v.at[c * half + j]], add=True)
        return carry

    lax.fori_loop(0, half, step, None)
    plsc.subcore_barrier()

    @pl.when(s == 0)
    def _():
        pltpu.sync_copy(deg_sh, deg_out.at[c])


_sc_deg = pl.kernel(
    _sc_deg_body,
    out_type=(jax.ShapeDtypeStruct((NC, N_NODES, DEGW), jnp.float32),),
    mesh=_mesh,
    compiler_params=pltpu.CompilerParams(use_tc_tiling_on_sc=False),
    scratch_types=[
        pltpu.VMEM((NCH, CH), jnp.int32),
        pltpu.VMEM((CH, DEGW), jnp.float32),
        pltpu.VMEM_SHARED((N_NODES, DEGW), jnp.float32),
    ],
)


RB = 2000


def _tc_layer_body(aggp_ref, degp_ref, w_ref, b_ref, out_ref):
    agg = jnp.concatenate([aggp_ref[0], aggp_ref[1]], axis=1)
    deg = degp_ref[0] + degp_ref[1]
    deg0 = jnp.maximum(deg[:, 0:1], 1.0)
    h = agg / deg0
    acc = jnp.dot(h, w_ref[...], preferred_element_type=jnp.float32)
    h = jnp.maximum(acc + b_ref[...], 0.0)
    out_ref[0] = h[:, :DH]
    out_ref[1] = h[:, DH:]


def _tc_layer(aggp, degp, w, b):
    grid = N_NODES // RB
    return pl.pallas_call(
        _tc_layer_body,
        grid=(grid,),
        in_specs=[
            pl.BlockSpec((NC, RB, DH), lambda i: (0, i, 0)),
            pl.BlockSpec((NC, RB, DEGW), lambda i: (0, i, 0)),
            pl.BlockSpec((D, D), lambda i: (0, 0)),
            pl.BlockSpec((1, D), lambda i: (0, 0)),
        ],
        out_specs=pl.BlockSpec((NC, RB, DH), lambda i: (0, i, 0)),
        out_shape=jax.ShapeDtypeStruct((NC, N_NODES, DH), jnp.float32),
    )(aggp, degp, w, b)


def _tc_final_body(aggp_ref, degp_ref, w3_ref, b3_ref, wf1_ref, bf1_ref,
                   wf2_ref, bf2_ref, out_ref):
    agg = jnp.concatenate([aggp_ref[0], aggp_ref[1]], axis=1)
    deg = degp_ref[0] + degp_ref[1]
    deg0 = jnp.maximum(deg[:, 0:1], 1.0)
    h = agg / deg0
    h = jnp.maximum(
        jnp.dot(h, w3_ref[...], preferred_element_type=jnp.float32)
        + b3_ref[...], 0.0)
    h = jnp.maximum(
        jnp.dot(h, wf1_ref[...], preferred_element_type=jnp.float32)
        + bf1_ref[...], 0.0)
    logits = (jnp.dot(h, wf2_ref[...], preferred_element_type=jnp.float32)
              + bf2_ref[...])
    col = lax.broadcasted_iota(jnp.int32, logits.shape, 1)
    logits = jnp.where(col < OUT, logits, -1e30)
    m = jnp.max(logits, axis=1, keepdims=True)
    e = jnp.exp(logits - m)
    lse = jnp.log(jnp.sum(e, axis=1, keepdims=True))
    out_ref[...] = logits - m - lse


def _tc_final(aggp, degp, w3, b3, wf1, bf1, wf2p, bf2p):
    grid = N_NODES // RB
    full = lambda r, c_: pl.BlockSpec((r, c_), lambda i: (0, 0))
    return pl.pallas_call(
        _tc_final_body,
        grid=(grid,),
        in_specs=[
            pl.BlockSpec((NC, RB, DH), lambda i: (0, i, 0)),
            pl.BlockSpec((NC, RB, DEGW), lambda i: (0, i, 0)),
            full(D, D), full(1, D),
            full(D, D), full(1, D),
            full(D, D), full(1, D),
        ],
        out_specs=pl.BlockSpec((RB, D), lambda i: (i, 0)),
        out_shape=jax.ShapeDtypeStruct((N_NODES, D), jnp.float32),
    )(aggp, degp, w3, b3, wf1, bf1, wf2p, bf2p)


def kernel(x, edge_index, W1, b1, W2, b2, W3, b3, Wf1, bf1, Wf2, bf2):
    ei = edge_index.astype(jnp.int32)
    src = ei[0].reshape(NS, NCH, CH)
    dst = ei[1].reshape(NS, NCH, CH)
    zagg = jnp.zeros((N_NODES, DH), jnp.float32)
    zdeg = jnp.zeros((N_NODES, DEGW), jnp.float32)
    ones = jnp.ones((CH, DEGW), jnp.float32)
    x2 = jnp.stack([x[:, :DH], x[:, DH:]])

    (degp,) = _sc_deg(dst, zdeg, ones)
    zagg, degp = jax.lax.optimization_barrier((zagg, degp))
    (aggp,) = _sc_agg(x2, src, dst, zagg)
    h2 = _tc_layer(aggp, degp, W1, b1.reshape(1, D))
    (aggp,) = _sc_agg(h2, src, dst, zagg)
    h2 = _tc_layer(aggp, degp, W2, b2.reshape(1, D))
    (aggp,) = _sc_agg(h2, src, dst, zagg)

    wf2p = jnp.zeros((D, D), jnp.float32).at[:, :OUT].set(Wf2)
    bf2p = jnp.zeros((1, D), jnp.float32).at[0, :OUT].set(bf2)
    out = _tc_final(aggp, degp, W3, b3.reshape(1, D),
                    Wf1, bf1.reshape(1, D), wf2p, bf2p)
    return out[:, :OUT]
